# Initial kernel scaffold; baseline (speedup 1.0000x reference)
import jax
import jax.numpy as jnp
from jax.experimental import pallas as pl


def kernel(features, labels, mask, edge_index, W1, b1, W2, b2):
    raise NotImplementedError("write your pallas kernel here")


# speedup vs baseline: 10.1174x; 10.1174x over previous
import functools

import jax
import jax.numpy as jnp
from jax import lax
from jax.experimental import pallas as pl
from jax.experimental.pallas import tpu as pltpu
from jax.experimental.pallas import tpu_sc as plsc

NW = 32
CH = 128


def _mesh():
    return plsc.VectorSubcoreMesh(core_axis_name="c", subcore_axis_name="s")


def _degrees_call(sidx_a, sidx_b, NP, NB):
    stripe = NP // 16
    sb16 = -(-stripe // 16) * 16

    @functools.partial(
        pl.kernel,
        out_type=tuple(jax.ShapeDtypeStruct((2 * NP,), jnp.float32)
                       for _ in range(2)),
        mesh=_mesh(),
        compiler_params=pltpu.CompilerParams(use_tc_tiling_on_sc=False),
        scratch_types=[
            pltpu.VMEM((NB, CH), jnp.int32),
            pltpu.VMEM((NB, CH), jnp.int32),
            pltpu.VMEM((CH,), jnp.float32),
            pltpu.VMEM((sb16,), jnp.float32),
            pltpu.VMEM_SHARED((NP,), jnp.float32),
            pltpu.VMEM_SHARED((NP,), jnp.float32),
        ],
    )
    def k(sa_h, sb_h, outa_h, outb_h, sva, svb, ones, tbuf, acca, accb):
        c = lax.axis_index("c")
        s = lax.axis_index("s")
        wid = s * 2 + c
        sl = pl.ds(s * stripe, stripe)
        osl = pl.ds(c * NP + s * stripe, stripe)
        tsl = pl.ds(0, stripe)
        pltpu.sync_copy(sa_h.at[wid], sva)
        pltpu.sync_copy(sb_h.at[wid], svb)
        for i in range(CH // 16):
            ones[pl.ds(i * 16, 16)] = jnp.ones((16,), jnp.float32)
        for i in range(sb16 // 16):
            tbuf[pl.ds(i * 16, 16)] = jnp.zeros((16,), jnp.float32)
        pltpu.sync_copy(tbuf.at[tsl], acca.at[sl])
        pltpu.sync_copy(tbuf.at[tsl], accb.at[sl])
        plsc.subcore_barrier()

        def body(j, carry):
            pltpu.sync_copy(ones, acca.at[sva.at[j]], add=True)
            pltpu.sync_copy(ones, accb.at[svb.at[j]], add=True)
            return carry

        lax.fori_loop(0, NB, body, 0)
        plsc.subcore_barrier()
        pltpu.sync_copy(acca.at[sl], tbuf.at[tsl])
        pltpu.sync_copy(tbuf.at[tsl], outa_h.at[osl])
        pltpu.sync_copy(accb.at[sl], tbuf.at[tsl])
        pltpu.sync_copy(tbuf.at[tsl], outb_h.at[osl])

    return k(sidx_a, sidx_b)


def _edge_agg_call(table, gidx, sidx, zeros, NP, D, NB):
    @functools.partial(
        pl.kernel,
        out_type=jax.ShapeDtypeStruct((2, NP, D), jnp.float32),
        mesh=_mesh(),
        compiler_params=pltpu.CompilerParams(use_tc_tiling_on_sc=False),
        scratch_types=[
            pltpu.VMEM((NB, CH), jnp.int32),
            pltpu.VMEM((NB, CH), jnp.int32),
            pltpu.VMEM((CH, D), jnp.float32),
            pltpu.VMEM_SHARED((NP, D), jnp.float32),
            pltpu.SemaphoreType.DMA,
        ],
    )
    def k(table_h, gidx_h, sidx_h, z_h, out_h, gv, sv, rows, acc, sem):
        c = lax.axis_index("c")
        s = lax.axis_index("s")
        wid = s * 2 + c
        stripe = NP // 16
        sl = pl.ds(s * stripe, stripe)
        pltpu.sync_copy(gidx_h.at[wid], gv)
        pltpu.sync_copy(sidx_h.at[wid], sv)
        pltpu.sync_copy(z_h.at[sl], acc.at[sl])
        plsc.subcore_barrier()

        def body(j, carry):
            pltpu.async_copy(table_h.at[gv.at[j]], rows, sem).wait()
            pltpu.sync_copy(rows, acc.at[sv.at[j]], add=True)
            return carry

        lax.fori_loop(0, NB, body, 0)
        plsc.subcore_barrier()
        pltpu.sync_copy(acc.at[sl], out_h.at[c, sl])

    return k(table, gidx, sidx, zeros)


def _mm1_call(x, W, d0, d1):
    N, Din = x.shape
    Dout = W.shape[1]
    R = 1000

    def body(x_r, w_r, d0_r, d1_r, o_r):
        ns = lax.rsqrt(jnp.maximum(d0_r[...] + d1_r[...], 1.0))
        o_r[...] = jnp.dot(x_r[...] * ns, w_r[...],
                           preferred_element_type=jnp.float32)

    return pl.pallas_call(
        body,
        grid=(N // R,),
        in_specs=[
            pl.BlockSpec((R, Din), lambda i: (i, 0)),
            pl.BlockSpec((Din, Dout), lambda i: (0, 0)),
            pl.BlockSpec((R, 1), lambda i: (i, 0)),
            pl.BlockSpec((R, 1), lambda i: (i, 0)),
        ],
        out_specs=pl.BlockSpec((R, Dout), lambda i: (i, 0)),
        out_shape=jax.ShapeDtypeStruct((N, Dout), jnp.float32),
    )(x, W, d0, d1)


def _mm2_call(p, dd0, dd1, ds0, ds1, b1r, W2, NP):
    N = dd0.shape[0]
    Din = p.shape[2]
    Dout = W2.shape[1]
    R = 1000

    def body(p0_r, p1_r, dd0_r, dd1_r, ds0_r, ds1_r, b_r, w_r, o_r):
        nd = lax.rsqrt(jnp.maximum(dd0_r[...] + dd1_r[...], 1.0))
        ns = lax.rsqrt(jnp.maximum(ds0_r[...] + ds1_r[...], 1.0))
        x1 = jnp.maximum((p0_r[0] + p1_r[0]) * nd + b_r[...], 0.0)
        o_r[...] = jnp.dot(x1, w_r[...], preferred_element_type=jnp.float32) * ns

    return pl.pallas_call(
        body,
        grid=(N // R,),
        in_specs=[
            pl.BlockSpec((1, R, Din), lambda i: (0, i, 0)),
            pl.BlockSpec((1, R, Din), lambda i: (1, i, 0)),
            pl.BlockSpec((R, 1), lambda i: (i, 0)),
            pl.BlockSpec((R, 1), lambda i: (i, 0)),
            pl.BlockSpec((R, 1), lambda i: (i, 0)),
            pl.BlockSpec((R, 1), lambda i: (i, 0)),
            pl.BlockSpec((1, Din), lambda i: (0, 0)),
            pl.BlockSpec((Din, Dout), lambda i: (0, 0)),
        ],
        out_specs=pl.BlockSpec((R, Dout), lambda i: (i, 0)),
        out_shape=jax.ShapeDtypeStruct((N, Dout), jnp.float32),
    )(p, p, dd0, dd1, ds0, ds1, b1r, W2)


def _fin_call(q, dd0, dd1, b2r, labels, maskf, NP):
    N, D = labels.shape
    R = 1000

    def body(q0_r, q1_r, dd0_r, dd1_r, b_r, lab_r, m_r, x_r, ml_r, nm_r):
        nd = lax.rsqrt(jnp.maximum(dd0_r[...] + dd1_r[...], 1.0))
        x_r[...] = (q0_r[0] + q1_r[0]) * nd + b_r[...]
        ml_r[...] = lab_r[...] * m_r[...]
        nm_r[...] = 1.0 - m_r[...]

    return pl.pallas_call(
        body,
        grid=(N // R,),
        in_specs=[
            pl.BlockSpec((1, R, D), lambda i: (0, i, 0)),
            pl.BlockSpec((1, R, D), lambda i: (1, i, 0)),
            pl.BlockSpec((R, 1), lambda i: (i, 0)),
            pl.BlockSpec((R, 1), lambda i: (i, 0)),
            pl.BlockSpec((1, D), lambda i: (0, 0)),
            pl.BlockSpec((R, D), lambda i: (i, 0)),
            pl.BlockSpec((R, 1), lambda i: (i, 0)),
        ],
        out_specs=[
            pl.BlockSpec((R, D), lambda i: (i, 0)),
            pl.BlockSpec((R, D), lambda i: (i, 0)),
            pl.BlockSpec((R, 1), lambda i: (i, 0)),
        ],
        out_shape=[
            jax.ShapeDtypeStruct((N, D), jnp.float32),
            jax.ShapeDtypeStruct((N, D), jnp.float32),
            jax.ShapeDtypeStruct((N, 1), jnp.float32),
        ],
    )(q, q, dd0, dd1, b2r, labels, maskf)


def _blend_call(r, nm, ml, NP):
    N, D = ml.shape
    R = 2000

    def body(r0_r, r1_r, nm_r, ml_r, y_r):
        y_r[...] = (r0_r[0] + r1_r[0]) * nm_r[...] + ml_r[...]

    return pl.pallas_call(
        body,
        grid=(N // R,),
        in_specs=[
            pl.BlockSpec((1, R, D), lambda i: (0, i, 0)),
            pl.BlockSpec((1, R, D), lambda i: (1, i, 0)),
            pl.BlockSpec((R, 1), lambda i: (i, 0)),
            pl.BlockSpec((R, D), lambda i: (i, 0)),
        ],
        out_specs=pl.BlockSpec((R, D), lambda i: (i, 0)),
        out_shape=jax.ShapeDtypeStruct((N, D), jnp.float32),
    )(r, r, nm, ml)


def kernel(features, labels, mask, edge_index, W1, b1, W2, b2):
    N, Din = features.shape
    Dh = W1.shape[1]
    Do = W2.shape[1]
    E = edge_index.shape[1]

    NB = -(-E // (NW * CH * 2)) * 2
    EP = NW * CH * NB
    NP = (N // 128 + 1) * 128
    PADR = NP - N

    src = edge_index[0].astype(jnp.int32)
    dst = edge_index[1].astype(jnp.int32)

    npad = EP - E
    ar = jnp.arange(npad, dtype=jnp.int32)
    gfill = ar % PADR
    sfill = N + ar % PADR

    def shape_idx(v, fill):
        return jnp.concatenate([v, fill]).reshape(NW, NB, CH)

    g_gcn = shape_idx(src, gfill)
    s_gcn = shape_idx(dst, sfill)
    g_lpa = shape_idx(dst, gfill)
    s_lpa = shape_idx(src, sfill)

    zeros1 = jnp.zeros((NP,), jnp.float32)
    zeros_h = jnp.zeros((NP, Dh), jnp.float32)
    zeros_o = jnp.zeros((NP, Do), jnp.float32)

    dega, degb = _degrees_call(s_lpa, s_gcn, NP, NB)
    ds0 = dega[:N].reshape(N, 1)
    ds1 = dega[NP:NP + N].reshape(N, 1)
    dd0 = degb[:N].reshape(N, 1)
    dd1 = degb[NP:NP + N].reshape(N, 1)

    b1r = b1.reshape(1, Dh)
    b2r = b2.reshape(1, Do)
    maskf = mask.astype(jnp.float32).reshape(N, 1)

    h1n = _mm1_call(features, W1, ds0, ds1)
    p = _edge_agg_call(h1n, g_gcn, s_gcn, zeros_h, NP, Dh, NB)
    h2n = _mm2_call(p, dd0, dd1, ds0, ds1, b1r, W2, NP)
    q = _edge_agg_call(h2n, g_gcn, s_gcn, zeros_o, NP, Do, NB)
    x2, ml, nm = _fin_call(q, dd0, dd1, b2r, labels, maskf, NP)

    y = ml
    for _ in range(10):
        r = _edge_agg_call(y, g_lpa, s_lpa, zeros_o, NP, Do, NB)
        y = _blend_call(r, nm, ml, NP)

    return (x2, y)


# speedup vs baseline: 17.6185x; 1.7414x over previous
import functools

import jax
import jax.numpy as jnp
from jax import lax
from jax.experimental import pallas as pl
from jax.experimental.pallas import tpu as pltpu
from jax.experimental.pallas import tpu_sc as plsc

NW = 32
CH = 128


def _mesh():
    return plsc.VectorSubcoreMesh(core_axis_name="c", subcore_axis_name="s")


def _degrees_call(sidx_a, sidx_b, NP, EW):
    stripe = NP // 16
    SCN = EW // 1024

    @functools.partial(
        pl.kernel,
        out_type=tuple(jax.ShapeDtypeStruct((2 * NP,), jnp.float32)
                       for _ in range(2)),
        mesh=_mesh(),
        compiler_params=pltpu.CompilerParams(use_tc_tiling_on_sc=False),
        scratch_types=[
            pltpu.VMEM((EW,), jnp.int32),
            pltpu.VMEM((EW,), jnp.int32),
            pltpu.VMEM((1024,), jnp.float32),
            pltpu.VMEM((stripe,), jnp.float32),
            pltpu.VMEM_SHARED((NP,), jnp.float32),
            pltpu.VMEM_SHARED((NP,), jnp.float32),
            pltpu.SemaphoreType.DMA,
            pltpu.SemaphoreType.DMA,
        ],
    )
    def k(sa_h, sb_h, outa_h, outb_h, sva, svb, ones, tbuf, acca, accb,
          sema, semb):
        c = lax.axis_index("c")
        s = lax.axis_index("s")
        wid = s * 2 + c
        sl = pl.ds(s * stripe, stripe)
        osl = pl.ds(c * NP + s * stripe, stripe)
        tsl = pl.ds(0, stripe)
        pltpu.sync_copy(sa_h.at[wid], sva)
        pltpu.sync_copy(sb_h.at[wid], svb)
        for i in range(1024 // 16):
            ones[pl.ds(i * 16, 16)] = jnp.ones((16,), jnp.float32)
        nz = stripe // 16
        for i in range(nz):
            tbuf[pl.ds(i * 16, 16)] = jnp.zeros((16,), jnp.float32)
        if stripe % 16:
            tbuf[pl.ds(stripe - 16, 16)] = jnp.zeros((16,), jnp.float32)
        pltpu.sync_copy(tbuf.at[tsl], acca.at[sl])
        pltpu.sync_copy(tbuf.at[tsl], accb.at[sl])
        plsc.subcore_barrier()

        cps = []
        for m in range(SCN):
            msl = pl.ds(m * 1024, 1024)
            cps.append(pltpu.async_copy(
                ones, acca.at[sva.at[msl]], sema, add=True))
            cps.append(pltpu.async_copy(
                ones, accb.at[svb.at[msl]], semb, add=True))
        for cp in cps:
            cp.wait()
        plsc.subcore_barrier()
        pltpu.sync_copy(acca.at[sl], tbuf.at[tsl])
        pltpu.sync_copy(tbuf.at[tsl], outa_h.at[osl])
        pltpu.sync_copy(accb.at[sl], tbuf.at[tsl])
        pltpu.sync_copy(tbuf.at[tsl], outb_h.at[osl])

    return k(sidx_a, sidx_b)


def _edge_agg_call(table, gidx, sidx, zeros, NP, D, EW, KC):
    NS = EW // KC

    @functools.partial(
        pl.kernel,
        out_type=jax.ShapeDtypeStruct((2, NP, D), jnp.float32),
        mesh=_mesh(),
        compiler_params=pltpu.CompilerParams(use_tc_tiling_on_sc=False),
        scratch_types=[
            pltpu.VMEM((EW,), jnp.int32),
            pltpu.VMEM((EW,), jnp.int32),
            pltpu.VMEM((KC, D), jnp.float32),
            pltpu.VMEM((KC, D), jnp.float32),
            pltpu.VMEM_SHARED((NP, D), jnp.float32),
            pltpu.SemaphoreType.DMA,
            pltpu.SemaphoreType.DMA,
            pltpu.SemaphoreType.DMA,
            pltpu.SemaphoreType.DMA,
        ],
    )
    def k(table_h, gidx_h, sidx_h, z_h, out_h, gv, sv, buf0, buf1, acc,
          gs0, gs1, ss0, ss1):
        c = lax.axis_index("c")
        s = lax.axis_index("s")
        wid = s * 2 + c
        stripe = NP // 16
        sl = pl.ds(s * stripe, stripe)
        pltpu.sync_copy(gidx_h.at[wid], gv)
        pltpu.sync_copy(sidx_h.at[wid], sv)
        pltpu.sync_copy(z_h.at[sl], acc.at[sl])
        plsc.subcore_barrier()

        bufs = (buf0, buf1)
        gsems = (gs0, gs1)
        ssems = (ss0, ss1)

        def gsl(m):
            return gv.at[pl.ds(m * KC, KC)]

        def ssl(m):
            return sv.at[pl.ds(m * KC, KC)]

        if NS <= 12:
            g = [None] * NS
            sc = [None] * NS
            for m in range(NS):
                b = m % 2
                if m >= 2:
                    sc[m - 2].wait()
                g[m] = pltpu.async_copy(table_h.at[gsl(m)], bufs[b], gsems[b])
                if m >= 1:
                    g[m - 1].wait()
                    sc[m - 1] = pltpu.async_copy(
                        bufs[1 - b], acc.at[ssl(m - 1)], ssems[1 - b], add=True)
            b = (NS - 1) % 2
            g[NS - 1].wait()
            sc[NS - 1] = pltpu.async_copy(
                bufs[b], acc.at[ssl(NS - 1)], ssems[b], add=True)
            if NS >= 2:
                sc[NS - 2].wait()
            sc[NS - 1].wait()
        else:
            def body(m2, carry):
                m = m2 * 2
                ga = pltpu.async_copy(table_h.at[gsl(m)], buf0, gs0)
                ga.wait()
                gb = pltpu.async_copy(table_h.at[gsl(m + 1)], buf1, gs1)
                sa = pltpu.async_copy(buf0, acc.at[ssl(m)], ss0, add=True)
                gb.wait()
                sa.wait()
                sb = pltpu.async_copy(buf1, acc.at[ssl(m + 1)], ss1, add=True)
                sb.wait()
                return carry

            lax.fori_loop(0, NS // 2, body, 0)

        plsc.subcore_barrier()
        pltpu.sync_copy(acc.at[sl], out_h.at[c, sl])

    return k(table, gidx, sidx, zeros)


def _mm1_call(x, W, d0, d1):
    N, Din = x.shape
    Dout = W.shape[1]
    R = 1000

    def body(x_r, w_r, d0_r, d1_r, o_r):
        ns = lax.rsqrt(jnp.maximum(d0_r[...] + d1_r[...], 1.0))
        o_r[...] = jnp.dot(x_r[...] * ns, w_r[...],
                           preferred_element_type=jnp.float32)

    return pl.pallas_call(
        body,
        grid=(N // R,),
        in_specs=[
            pl.BlockSpec((R, Din), lambda i: (i, 0)),
            pl.BlockSpec((Din, Dout), lambda i: (0, 0)),
            pl.BlockSpec((R, 1), lambda i: (i, 0)),
            pl.BlockSpec((R, 1), lambda i: (i, 0)),
        ],
        out_specs=pl.BlockSpec((R, Dout), lambda i: (i, 0)),
        out_shape=jax.ShapeDtypeStruct((N, Dout), jnp.float32),
    )(x, W, d0, d1)


def _mm2_call(p, dd0, dd1, ds0, ds1, b1r, W2, NP):
    N = dd0.shape[0]
    Din = p.shape[2]
    Dout = W2.shape[1]
    R = 1000

    def body(p0_r, p1_r, dd0_r, dd1_r, ds0_r, ds1_r, b_r, w_r, o_r):
        nd = lax.rsqrt(jnp.maximum(dd0_r[...] + dd1_r[...], 1.0))
        ns = lax.rsqrt(jnp.maximum(ds0_r[...] + ds1_r[...], 1.0))
        x1 = jnp.maximum((p0_r[0] + p1_r[0]) * nd + b_r[...], 0.0)
        o_r[...] = jnp.dot(x1, w_r[...], preferred_element_type=jnp.float32) * ns

    return pl.pallas_call(
        body,
        grid=(N // R,),
        in_specs=[
            pl.BlockSpec((1, R, Din), lambda i: (0, i, 0)),
            pl.BlockSpec((1, R, Din), lambda i: (1, i, 0)),
            pl.BlockSpec((R, 1), lambda i: (i, 0)),
            pl.BlockSpec((R, 1), lambda i: (i, 0)),
            pl.BlockSpec((R, 1), lambda i: (i, 0)),
            pl.BlockSpec((R, 1), lambda i: (i, 0)),
            pl.BlockSpec((1, Din), lambda i: (0, 0)),
            pl.BlockSpec((Din, Dout), lambda i: (0, 0)),
        ],
        out_specs=pl.BlockSpec((R, Dout), lambda i: (i, 0)),
        out_shape=jax.ShapeDtypeStruct((N, Dout), jnp.float32),
    )(p, p, dd0, dd1, ds0, ds1, b1r, W2)


def _fin_call(q, dd0, dd1, b2r, labels, maskf, NP):
    N, D = labels.shape
    R = 1000

    def body(q0_r, q1_r, dd0_r, dd1_r, b_r, lab_r, m_r, x_r, ml_r, nm_r):
        nd = lax.rsqrt(jnp.maximum(dd0_r[...] + dd1_r[...], 1.0))
        x_r[...] = (q0_r[0] + q1_r[0]) * nd + b_r[...]
        ml_r[...] = lab_r[...] * m_r[...]
        nm_r[...] = 1.0 - m_r[...]

    return pl.pallas_call(
        body,
        grid=(N // R,),
        in_specs=[
            pl.BlockSpec((1, R, D), lambda i: (0, i, 0)),
            pl.BlockSpec((1, R, D), lambda i: (1, i, 0)),
            pl.BlockSpec((R, 1), lambda i: (i, 0)),
            pl.BlockSpec((R, 1), lambda i: (i, 0)),
            pl.BlockSpec((1, D), lambda i: (0, 0)),
            pl.BlockSpec((R, D), lambda i: (i, 0)),
            pl.BlockSpec((R, 1), lambda i: (i, 0)),
        ],
        out_specs=[
            pl.BlockSpec((R, D), lambda i: (i, 0)),
            pl.BlockSpec((R, D), lambda i: (i, 0)),
            pl.BlockSpec((R, 1), lambda i: (i, 0)),
        ],
        out_shape=[
            jax.ShapeDtypeStruct((N, D), jnp.float32),
            jax.ShapeDtypeStruct((N, D), jnp.float32),
            jax.ShapeDtypeStruct((N, 1), jnp.float32),
        ],
    )(q, q, dd0, dd1, b2r, labels, maskf)


def _blend_call(r, nm, ml, NP):
    N, D = ml.shape
    R = 2000

    def body(r0_r, r1_r, nm_r, ml_r, y_r):
        y_r[...] = (r0_r[0] + r1_r[0]) * nm_r[...] + ml_r[...]

    return pl.pallas_call(
        body,
        grid=(N // R,),
        in_specs=[
            pl.BlockSpec((1, R, D), lambda i: (0, i, 0)),
            pl.BlockSpec((1, R, D), lambda i: (1, i, 0)),
            pl.BlockSpec((R, 1), lambda i: (i, 0)),
            pl.BlockSpec((R, D), lambda i: (i, 0)),
        ],
        out_specs=pl.BlockSpec((R, D), lambda i: (i, 0)),
        out_shape=jax.ShapeDtypeStruct((N, D), jnp.float32),
    )(r, r, nm, ml)


def kernel(features, labels, mask, edge_index, W1, b1, W2, b2):
    N, Din = features.shape
    Dh = W1.shape[1]
    Do = W2.shape[1]
    E = edge_index.shape[1]

    NB = -(-E // (NW * CH * 2)) * 2
    EP = NW * CH * NB
    NP = (N // 128 + 1) * 128
    PADR = NP - N

    src = edge_index[0].astype(jnp.int32)
    dst = edge_index[1].astype(jnp.int32)

    npad = EP - E
    ar = jnp.arange(npad, dtype=jnp.int32)
    gfill = ar % PADR
    sfill = N + ar % PADR

    def shape_idx(v, fill):
        return jnp.concatenate([v, fill]).reshape(NW, NB * CH)

    g_gcn = shape_idx(src, gfill)
    s_gcn = shape_idx(dst, sfill)
    g_lpa = shape_idx(dst, gfill)
    s_lpa = shape_idx(src, sfill)

    zeros1 = jnp.zeros((NP,), jnp.float32)
    zeros_h = jnp.zeros((NP, Dh), jnp.float32)
    zeros_o = jnp.zeros((NP, Do), jnp.float32)

    dega, degb = _degrees_call(s_lpa, s_gcn, NP, NB * CH)
    ds0 = dega[:N].reshape(N, 1)
    ds1 = dega[NP:NP + N].reshape(N, 1)
    dd0 = degb[:N].reshape(N, 1)
    dd1 = degb[NP:NP + N].reshape(N, 1)

    b1r = b1.reshape(1, Dh)
    b2r = b2.reshape(1, Do)
    maskf = mask.astype(jnp.float32).reshape(N, 1)

    h1n = _mm1_call(features, W1, ds0, ds1)
    p = _edge_agg_call(h1n, g_gcn, s_gcn, zeros_h, NP, Dh, NB * CH, 80)
    h2n = _mm2_call(p, dd0, dd1, ds0, ds1, b1r, W2, NP)
    q = _edge_agg_call(h2n, g_gcn, s_gcn, zeros_o, NP, Do, NB * CH, 2048)
    x2, ml, nm = _fin_call(q, dd0, dd1, b2r, labels, maskf, NP)

    y = ml
    for _ in range(10):
        r = _edge_agg_call(y, g_lpa, s_lpa, zeros_o, NP, Do, NB * CH, 2048)
        y = _blend_call(r, nm, ml, NP)

    return (x2, y)


# speedup vs baseline: 25.9348x; 1.4720x over previous
import functools

import jax
import jax.numpy as jnp
from jax import lax
from jax.experimental import pallas as pl
from jax.experimental.pallas import tpu as pltpu
from jax.experimental.pallas import tpu_sc as plsc

NW = 32
CH = 128


def _mesh():
    return plsc.VectorSubcoreMesh(core_axis_name="c", subcore_axis_name="s")


def _degrees_call(sidx_a, sidx_b, NP, EW):
    stripe = NP // 16
    SCN = EW // 1024

    @functools.partial(
        pl.kernel,
        out_type=tuple(jax.ShapeDtypeStruct((2 * NP,), jnp.float32)
                       for _ in range(2)),
        mesh=_mesh(),
        compiler_params=pltpu.CompilerParams(use_tc_tiling_on_sc=False),
        scratch_types=[
            pltpu.VMEM((EW,), jnp.int32),
            pltpu.VMEM((EW,), jnp.int32),
            pltpu.VMEM((1024,), jnp.float32),
            pltpu.VMEM((stripe,), jnp.float32),
            pltpu.VMEM_SHARED((NP,), jnp.float32),
            pltpu.VMEM_SHARED((NP,), jnp.float32),
            pltpu.SemaphoreType.DMA,
            pltpu.SemaphoreType.DMA,
        ],
    )
    def k(sa_h, sb_h, outa_h, outb_h, sva, svb, ones, tbuf, acca, accb,
          sema, semb):
        c = lax.axis_index("c")
        s = lax.axis_index("s")
        wid = s * 2 + c
        sl = pl.ds(s * stripe, stripe)
        osl = pl.ds(c * NP + s * stripe, stripe)
        tsl = pl.ds(0, stripe)
        pltpu.sync_copy(sa_h.at[wid], sva)
        pltpu.sync_copy(sb_h.at[wid], svb)
        for i in range(1024 // 16):
            ones[pl.ds(i * 16, 16)] = jnp.ones((16,), jnp.float32)
        nz = stripe // 16
        for i in range(nz):
            tbuf[pl.ds(i * 16, 16)] = jnp.zeros((16,), jnp.float32)
        if stripe % 16:
            tbuf[pl.ds(stripe - 16, 16)] = jnp.zeros((16,), jnp.float32)
        pltpu.sync_copy(tbuf.at[tsl], acca.at[sl])
        pltpu.sync_copy(tbuf.at[tsl], accb.at[sl])
        plsc.subcore_barrier()

        cps = []
        for m in range(SCN):
            msl = pl.ds(m * 1024, 1024)
            cps.append(pltpu.async_copy(
                ones, acca.at[sva.at[msl]], sema, add=True))
            cps.append(pltpu.async_copy(
                ones, accb.at[svb.at[msl]], semb, add=True))
        for cp in cps:
            cp.wait()
        plsc.subcore_barrier()
        pltpu.sync_copy(acca.at[sl], tbuf.at[tsl])
        pltpu.sync_copy(tbuf.at[tsl], outa_h.at[osl])
        pltpu.sync_copy(accb.at[sl], tbuf.at[tsl])
        pltpu.sync_copy(tbuf.at[tsl], outb_h.at[osl])

    return k(sidx_a, sidx_b)


def _edge_agg_call(table, gidx, sidx, zeros, NP, D, EW, KC):
    NS = EW // KC

    @functools.partial(
        pl.kernel,
        out_type=jax.ShapeDtypeStruct((2, NP, D), jnp.float32),
        mesh=_mesh(),
        compiler_params=pltpu.CompilerParams(use_tc_tiling_on_sc=False),
        scratch_types=[
            pltpu.VMEM((EW,), jnp.int32),
            pltpu.VMEM((EW,), jnp.int32),
            pltpu.VMEM((KC, D), jnp.float32),
            pltpu.VMEM((KC, D), jnp.float32),
            pltpu.VMEM_SHARED((NP, D), jnp.float32),
            pltpu.SemaphoreType.DMA,
            pltpu.SemaphoreType.DMA,
            pltpu.SemaphoreType.DMA,
            pltpu.SemaphoreType.DMA,
        ],
    )
    def k(table_h, gidx_h, sidx_h, z_h, out_h, gv, sv, buf0, buf1, acc,
          gs0, gs1, ss0, ss1):
        c = lax.axis_index("c")
        s = lax.axis_index("s")
        wid = s * 2 + c
        stripe = NP // 16
        sl = pl.ds(s * stripe, stripe)
        pltpu.sync_copy(gidx_h.at[wid], gv)
        pltpu.sync_copy(sidx_h.at[wid], sv)
        pltpu.sync_copy(z_h.at[sl], acc.at[sl])
        plsc.subcore_barrier()

        bufs = (buf0, buf1)
        gsems = (gs0, gs1)
        ssems = (ss0, ss1)

        def gsl(m):
            return gv.at[pl.ds(m * KC, KC)]

        def ssl(m):
            return sv.at[pl.ds(m * KC, KC)]

        if NS <= 12:
            g = [None] * NS
            sc = [None] * NS
            for m in range(NS):
                b = m % 2
                if m >= 2:
                    sc[m - 2].wait()
                g[m] = pltpu.async_copy(table_h.at[gsl(m)], bufs[b], gsems[b])
                if m >= 1:
                    g[m - 1].wait()
                    sc[m - 1] = pltpu.async_copy(
                        bufs[1 - b], acc.at[ssl(m - 1)], ssems[1 - b], add=True)
            b = (NS - 1) % 2
            g[NS - 1].wait()
            sc[NS - 1] = pltpu.async_copy(
                bufs[b], acc.at[ssl(NS - 1)], ssems[b], add=True)
            if NS >= 2:
                sc[NS - 2].wait()
            sc[NS - 1].wait()
        else:
            def body(m2, carry):
                m = m2 * 2
                ga = pltpu.async_copy(table_h.at[gsl(m)], buf0, gs0)
                ga.wait()
                gb = pltpu.async_copy(table_h.at[gsl(m + 1)], buf1, gs1)
                sa = pltpu.async_copy(buf0, acc.at[ssl(m)], ss0, add=True)
                gb.wait()
                sa.wait()
                sb = pltpu.async_copy(buf1, acc.at[ssl(m + 1)], ss1, add=True)
                sb.wait()
                return carry

            lax.fori_loop(0, NS // 2, body, 0)

        plsc.subcore_barrier()
        pltpu.sync_copy(acc.at[sl], out_h.at[c, sl])

    return k(table, gidx, sidx, zeros)


def _lpa_call(ml2, zeros8, gidx1, sidx1, ua1, NP, EWc, KC, ITERS):
    NS = EWc // KC
    stripe = NP // 16
    UL = stripe

    @functools.partial(
        pl.kernel,
        out_type=jax.ShapeDtypeStruct((2 * NP, 8), jnp.float32),
        mesh=_mesh(),
        compiler_params=pltpu.CompilerParams(use_tc_tiling_on_sc=False),
        scratch_types=[
            pltpu.VMEM((EWc,), jnp.int32),
            pltpu.VMEM((EWc,), jnp.int32),
            pltpu.VMEM((UL,), jnp.int32),
            pltpu.VMEM((KC, 8), jnp.float32),
            pltpu.VMEM((KC, 8), jnp.float32),
            pltpu.VMEM((stripe, 8), jnp.float32),
            pltpu.VMEM((UL, 8), jnp.float32),
            pltpu.VMEM_SHARED((NP, 8), jnp.float32),
            pltpu.VMEM_SHARED((NP, 8), jnp.float32),
            pltpu.SemaphoreType.DMA,
            pltpu.SemaphoreType.DMA,
            pltpu.SemaphoreType.DMA,
            pltpu.SemaphoreType.DMA,
        ],
    )
    def k(ml_h, z_h, gidx_h, sidx_h, ua_h, y_h,
          gv, sv, uav, buf0, buf1, zbuf, ubuf, acc, ysp,
          gs0, gs1, ss0, ss1):
        c = lax.axis_index("c")
        s = lax.axis_index("s")
        sl = pl.ds(s * stripe, stripe)
        osl = pl.ds(c * NP + s * stripe, stripe)
        pltpu.sync_copy(gidx_h.at[s], gv)
        pltpu.sync_copy(sidx_h.at[s], sv)
        pltpu.sync_copy(ua_h.at[s], uav)
        pltpu.sync_copy(z_h.at[sl], zbuf)
        pltpu.sync_copy(ml_h.at[osl], ubuf)
        pltpu.sync_copy(ubuf, ysp.at[sl])
        pltpu.sync_copy(zbuf, acc.at[sl])
        plsc.subcore_barrier()

        bufs = (buf0, buf1)
        gsems = (gs0, gs1)
        ssems = (ss0, ss1)
        for _ in range(ITERS):
            if True:
                g = [None] * NS
                sc = [None] * NS
                for m in range(NS):
                    b = m % 2
                    if m >= 2:
                        sc[m - 2].wait()
                    g[m] = pltpu.async_copy(
                        ysp.at[gv.at[pl.ds(m * KC, KC)]], bufs[b], gsems[b])
                    if m >= 1:
                        g[m - 1].wait()
                        sc[m - 1] = pltpu.async_copy(
                            bufs[1 - b], acc.at[sv.at[pl.ds((m - 1) * KC, KC)]],
                            ssems[1 - b], add=True)
                b = (NS - 1) % 2
                g[NS - 1].wait()
                sc[NS - 1] = pltpu.async_copy(
                    bufs[b], acc.at[sv.at[pl.ds((NS - 1) * KC, KC)]],
                    ssems[b], add=True)
                sc[NS - 2].wait()
                sc[NS - 1].wait()
            plsc.subcore_barrier()
            pltpu.async_copy(acc.at[uav], ubuf, gs0).wait()
            pltpu.async_copy(ubuf, ysp.at[uav], ss0).wait()
            plsc.subcore_barrier()
            pltpu.sync_copy(zbuf, acc.at[sl])
            plsc.subcore_barrier()

        pltpu.sync_copy(ysp.at[sl], ubuf)
        pltpu.sync_copy(ubuf, y_h.at[osl])

    return k(ml2, zeros8, gidx1, sidx1, ua1)


def _mm1_call(x, W, d0, d1):
    N, Din = x.shape
    Dout = W.shape[1]
    R = 1000

    def body(x_r, w_r, d0_r, d1_r, o_r):
        ns = lax.rsqrt(jnp.maximum(d0_r[...] + d1_r[...], 1.0))
        o_r[...] = jnp.dot(x_r[...] * ns, w_r[...],
                           preferred_element_type=jnp.float32)

    return pl.pallas_call(
        body,
        grid=(N // R,),
        in_specs=[
            pl.BlockSpec((R, Din), lambda i: (i, 0)),
            pl.BlockSpec((Din, Dout), lambda i: (0, 0)),
            pl.BlockSpec((R, 1), lambda i: (i, 0)),
            pl.BlockSpec((R, 1), lambda i: (i, 0)),
        ],
        out_specs=pl.BlockSpec((R, Dout), lambda i: (i, 0)),
        out_shape=jax.ShapeDtypeStruct((N, Dout), jnp.float32),
    )(x, W, d0, d1)


def _mm2_call(p, dd0, dd1, ds0, ds1, b1r, W2, NP):
    N = dd0.shape[0]
    Din = p.shape[2]
    Dout = W2.shape[1]
    R = 1000

    def body(p0_r, p1_r, dd0_r, dd1_r, ds0_r, ds1_r, b_r, w_r, o_r):
        nd = lax.rsqrt(jnp.maximum(dd0_r[...] + dd1_r[...], 1.0))
        ns = lax.rsqrt(jnp.maximum(ds0_r[...] + ds1_r[...], 1.0))
        x1 = jnp.maximum((p0_r[0] + p1_r[0]) * nd + b_r[...], 0.0)
        o_r[...] = jnp.dot(x1, w_r[...], preferred_element_type=jnp.float32) * ns

    return pl.pallas_call(
        body,
        grid=(N // R,),
        in_specs=[
            pl.BlockSpec((1, R, Din), lambda i: (0, i, 0)),
            pl.BlockSpec((1, R, Din), lambda i: (1, i, 0)),
            pl.BlockSpec((R, 1), lambda i: (i, 0)),
            pl.BlockSpec((R, 1), lambda i: (i, 0)),
            pl.BlockSpec((R, 1), lambda i: (i, 0)),
            pl.BlockSpec((R, 1), lambda i: (i, 0)),
            pl.BlockSpec((1, Din), lambda i: (0, 0)),
            pl.BlockSpec((Din, Dout), lambda i: (0, 0)),
        ],
        out_specs=pl.BlockSpec((R, Dout), lambda i: (i, 0)),
        out_shape=jax.ShapeDtypeStruct((N, Dout), jnp.float32),
    )(p, p, dd0, dd1, ds0, ds1, b1r, W2)


def _fin_call(q, dd0, dd1, b2r, labels, maskf, NP):
    N, D = labels.shape
    R = 1000

    def body(q0_r, q1_r, dd0_r, dd1_r, b_r, lab_r, m_r, x_r, ml_r, nm_r):
        nd = lax.rsqrt(jnp.maximum(dd0_r[...] + dd1_r[...], 1.0))
        x_r[...] = (q0_r[0] + q1_r[0]) * nd + b_r[...]
        ml_r[...] = lab_r[...] * m_r[...]
        nm_r[...] = 1.0 - m_r[...]

    return pl.pallas_call(
        body,
        grid=(N // R,),
        in_specs=[
            pl.BlockSpec((1, R, D), lambda i: (0, i, 0)),
            pl.BlockSpec((1, R, D), lambda i: (1, i, 0)),
            pl.BlockSpec((R, 1), lambda i: (i, 0)),
            pl.BlockSpec((R, 1), lambda i: (i, 0)),
            pl.BlockSpec((1, D), lambda i: (0, 0)),
            pl.BlockSpec((R, D), lambda i: (i, 0)),
            pl.BlockSpec((R, 1), lambda i: (i, 0)),
        ],
        out_specs=[
            pl.BlockSpec((R, D), lambda i: (i, 0)),
            pl.BlockSpec((R, D), lambda i: (i, 0)),
            pl.BlockSpec((R, 1), lambda i: (i, 0)),
        ],
        out_shape=[
            jax.ShapeDtypeStruct((N, D), jnp.float32),
            jax.ShapeDtypeStruct((N, D), jnp.float32),
            jax.ShapeDtypeStruct((N, 1), jnp.float32),
        ],
    )(q, q, dd0, dd1, b2r, labels, maskf)


def _blend_call(r, nm, ml, NP):
    N, D = ml.shape
    R = 2000

    def body(r0_r, r1_r, nm_r, ml_r, y_r):
        y_r[...] = (r0_r[0] + r1_r[0]) * nm_r[...] + ml_r[...]

    return pl.pallas_call(
        body,
        grid=(N // R,),
        in_specs=[
            pl.BlockSpec((1, R, D), lambda i: (0, i, 0)),
            pl.BlockSpec((1, R, D), lambda i: (1, i, 0)),
            pl.BlockSpec((R, 1), lambda i: (i, 0)),
            pl.BlockSpec((R, D), lambda i: (i, 0)),
        ],
        out_specs=pl.BlockSpec((R, D), lambda i: (i, 0)),
        out_shape=jax.ShapeDtypeStruct((N, D), jnp.float32),
    )(r, r, nm, ml)


def kernel(features, labels, mask, edge_index, W1, b1, W2, b2):
    N, Din = features.shape
    Dh = W1.shape[1]
    Do = W2.shape[1]
    E = edge_index.shape[1]

    NB = -(-E // (NW * CH * 2)) * 2
    EP = NW * CH * NB
    NP = (N // 128 + 1) * 128
    PADR = NP - N

    src = edge_index[0].astype(jnp.int32)
    dst = edge_index[1].astype(jnp.int32)

    npad = EP - E
    ar = jnp.arange(npad, dtype=jnp.int32)
    gfill = ar % PADR
    sfill = N + ar % PADR

    def shape_idx(v, fill):
        return jnp.concatenate([v, fill]).reshape(NW, NB * CH)

    g_gcn = shape_idx(src, gfill)
    s_gcn = shape_idx(dst, sfill)
    g_lpa = shape_idx(dst, gfill)
    s_lpa = shape_idx(src, sfill)

    zeros1 = jnp.zeros((NP,), jnp.float32)
    zeros_h = jnp.zeros((NP, Dh), jnp.float32)
    zeros_o = jnp.zeros((NP, Do), jnp.float32)

    dega, degb = _degrees_call(s_lpa, s_gcn, NP, NB * CH)
    ds0 = dega[:N].reshape(N, 1)
    ds1 = dega[NP:NP + N].reshape(N, 1)
    dd0 = degb[:N].reshape(N, 1)
    dd1 = degb[NP:NP + N].reshape(N, 1)

    b1r = b1.reshape(1, Dh)
    b2r = b2.reshape(1, Do)
    maskf = mask.astype(jnp.float32).reshape(N, 1)

    h1n = _mm1_call(features, W1, ds0, ds1)
    p = _edge_agg_call(h1n, g_gcn, s_gcn, zeros_h, NP, Dh, NB * CH, 80)
    h2n = _mm2_call(p, dd0, dd1, ds0, ds1, b1r, W2, NP)
    q = _edge_agg_call(h2n, g_gcn, s_gcn, zeros_o, NP, Do, NB * CH, 2048)
    x2, ml, nm = _fin_call(q, dd0, dd1, b2r, labels, maskf, NP)

    KC2 = 2048
    EWc = -(-E // (16 * KC2)) * KC2
    padE = 16 * EWc - E
    ar2 = jnp.arange(padE, dtype=jnp.int32)
    srcp = jnp.concatenate([src, N + ar2 % PADR])
    dstp = jnp.concatenate([dst, N + ar2 % PADR])
    sidx1 = srcp.reshape(16, EWc)
    gidx1 = dstp.reshape(16, EWc)

    UL = NP // 16
    ui = jnp.nonzero(jnp.logical_not(mask), size=N, fill_value=N)[0]
    ui = ui.astype(jnp.int32)
    ui = jnp.where(ui == N, N + jnp.arange(N, dtype=jnp.int32) % PADR, ui)
    uif = jnp.concatenate([ui, N + jnp.arange(NP - N, dtype=jnp.int32) % PADR])
    ua1 = uif.reshape(16, UL)

    mlp = jnp.pad(ml, ((0, NP - N), (0, 0)))
    ml2 = jnp.concatenate([mlp[:, :8], mlp[:, 8:]], axis=0)
    zeros8 = jnp.zeros((NP, 8), jnp.float32)

    yflat = _lpa_call(ml2, zeros8, gidx1, sidx1, ua1, NP, EWc, KC2, 10)
    y = jnp.concatenate([yflat[:N, :], yflat[NP:NP + N, :]], axis=1)

    return (x2, y)
